# baseline (speedup 1.0000x reference)
from math import sqrt

import jax
import jax.numpy as jnp
import numpy as np
from jax import lax
from jax.experimental import pallas as pl
from jax.experimental.pallas import tpu as pltpu
from jax.experimental.pallas import tpu_sc as plsc

_MAPS_SIZE = [38, 19, 10, 5, 3, 1]
_NUM_ANCHORS = [4, 6, 6, 6, 4, 4]
_RATIOS = [1.0, 2.0, 0.5, 3.0, 1.0 / 3.0]
_THRESHOLD = 0.5
_SCALE_NEG = 3


def _dbox_scale(k, m=6, smin=0.2, smax=0.9):
    return smin + (smax - smin) * (k - 1) / (m - 1)


def _default_box_np():
    m = 6
    scales = [_dbox_scale(k) for k in range(1, m + 2)]
    scales_hat = [sqrt(scales[k] * scales[k + 1]) for k in range(m)]
    boxes = []
    for k in range(m):
        size = _MAPS_SIZE[k]
        coords = (np.arange(size, dtype=np.float32) + 0.5) / size
        cx = np.tile(coords[None, :], (size, 1)).reshape(-1)
        cy = np.tile(coords[:, None], (1, size)).reshape(-1)
        for idx in range(_NUM_ANCHORS[k] - 1):
            w = np.full_like(cx, scales[k] * sqrt(_RATIOS[idx]))
            h = np.full_like(cx, scales[k] / sqrt(_RATIOS[idx]))
            boxes.append(np.stack([cx, cy, w, h], axis=0))
        s = scales_hat[k]
        boxes.append(np.stack([cx, cy, np.full_like(cx, s), np.full_like(cx, s)], axis=0))
    return np.concatenate(boxes, axis=1).astype(np.float32)


_DB = _default_box_np()
_A = _DB.shape[1]

_Z = np.zeros_like(_DB[0])
_O = np.ones_like(_DB[0])
_DB_PACK = np.stack(
    [
        _DB[0] - _DB[2] / 2, _DB[1] - _DB[3] / 2,
        _DB[0] + _DB[2] / 2, _DB[1] + _DB[3] / 2,
        _DB[2] * _DB[3] + 1e-9,
        _DB[0], _DB[1], _Z, _Z,
        1.0 / _DB[2], 1.0 / _DB[3], _O, _O,
        _Z, _Z, -np.log(_DB[2]), -np.log(_DB[3]),
    ],
    axis=0,
).astype(np.float32)
_THIRD = np.float32(1.0 / 3.0)


_BPP = 2


def _phase1_kernel(db_ref, y_ref, tb_ref, tbt_ref, lab_ref, keys_ref, part_ref, np_ref):
    b = pl.program_id(0)

    def one_batch(i):
        return _one_batch(db_ref, y_ref, tb_ref, tbt_ref, lab_ref, keys_ref, i)

    part_b, num_pos_b = one_batch(0)
    for i in range(1, _BPP):
        p, n = one_batch(i)
        part_b = part_b + p
        num_pos_b = num_pos_b + n

    @pl.when(b == 0)
    def _():
        part_ref[0] = part_b
        np_ref[0] = num_pos_b

    @pl.when(b != 0)
    def _():
        part_ref[0] = part_ref[0] + part_b
        np_ref[0] = np_ref[0] + num_pos_b


def _one_batch(db_ref, y_ref, tb_ref, tbt_ref, lab_ref, keys_ref, i):
    db_l = db_ref[0:1, :]
    db_t = db_ref[1:2, :]
    db_r = db_ref[2:3, :]
    db_b = db_ref[3:4, :]
    db_area = db_ref[4:5, :]

    tbt = tbt_ref[i]
    gcx = tbt[:, 0:1]
    gcy = tbt[:, 1:2]
    gw = tbt[:, 2:3]
    gh = tbt[:, 3:4]
    gl = gcx - gw * 0.5
    gt = gcy - gh * 0.5
    gr = gcx + gw * 0.5
    gb = gcy + gh * 0.5
    g_area = gw * gh

    il = jnp.maximum(gl, db_l)
    it = jnp.maximum(gt, db_t)
    ir = jnp.minimum(gr, db_r)
    ib = jnp.minimum(gb, db_b)
    inter = jnp.maximum(ir - il, 0.0) * jnp.maximum(ib - it, 0.0)
    r = inter / (g_area + db_area)

    genc = 31 - jax.lax.broadcasted_iota(jnp.int32, (32, 1), 0)
    ki = (jax.lax.bitcast_convert_type(r, jnp.int32) & jnp.int32(~31)) | genc
    best = jnp.max(ki, axis=0, keepdims=True)

    pos = jax.lax.bitcast_convert_type(best & jnp.int32(~31), jnp.float32) > _THIRD
    posf = pos.astype(jnp.float32)
    num_pos_b = jnp.sum(pos.astype(jnp.int32))

    maskf = (ki == best).astype(jnp.float32)
    tb = tb_ref[i]
    logw = jnp.log(tb[2:3, :])
    logh = jnp.log(tb[3:4, :])
    labf = lab_ref[i]
    zeros3 = jnp.zeros((3, 32), jnp.float32)
    stacked = jnp.concatenate([tb[0:1], tb[1:2], logw, logh, labf, zeros3], axis=0)
    mm = jax.lax.dot_general(
        stacked, maskf, (((1,), (0,)), ((), ())),
        preferred_element_type=jnp.float32,
    )
    blab = mm[4:5]

    gh = (mm[0:4] - db_ref[5:9, :]) * db_ref[9:13, :] + db_ref[13:17, :]
    d = y_ref[i, 0:4, :] - gh
    ad = jnp.abs(d)
    sl1 = jnp.where(ad < 1.0, 0.5 * d * d, ad - 0.5)
    loc_loss = jnp.sum(sl1 * posf)

    cls = y_ref[i, 4:25, :]
    m = jnp.max(cls, axis=0, keepdims=True)
    s = jnp.sum(jnp.exp(cls - m), axis=0, keepdims=True)
    lse = m + jnp.log(s)

    ciota = jax.lax.broadcasted_iota(jnp.int32, (21, 1), 0).astype(jnp.float32)
    chmask = (blab == ciota).astype(jnp.float32) * posf
    sel_sum = jnp.sum(chmask * cls)
    pos_loss = jnp.sum(lse * posf) - sel_sum

    neg = lse - y_ref[i, 4:5, :]
    keys = jnp.where(pos, jnp.int32(-1), jax.lax.bitcast_convert_type(neg, jnp.int32))
    keys_ref[i] = keys

    return loc_loss + pos_loss, num_pos_b


_NB = 32768
_SC_TILES = 32


def _sc_hist_kernel(keys_hbm, cnt_hbm, sum_hbm, chunk_v, hcnt_v, hsum_v, sem):
    wid = lax.axis_index("s") * 2 + lax.axis_index("c")
    chunk = keys_hbm.shape[0] // _SC_TILES
    cp = pltpu.async_copy(keys_hbm.at[pl.ds(wid * chunk, chunk)], chunk_v, sem)

    zi = jnp.zeros((16,), jnp.int32)
    zf = jnp.zeros((16,), jnp.float32)

    def zbody(j, c):
        hcnt_v[pl.ds(j * 16, 16)] = zi
        hsum_v[pl.ds(j * 16, 16)] = zf
        return c

    lax.fori_loop(0, _NB // 16, zbody, 0, unroll=8)
    cp.wait()

    ones = jnp.ones((16,), jnp.int32)

    def body(j, c):
        k16 = chunk_v[pl.ds(j * 16, 16)]
        msk = k16 >= 0
        bkt = lax.shift_right_logical(jnp.maximum(k16, 0), 16)
        plsc.addupdate_scatter(hcnt_v, [bkt], ones, mask=msk)
        plsc.addupdate_scatter(hsum_v, [bkt], plsc.bitcast(k16, jnp.float32), mask=msk)
        return c

    lax.fori_loop(0, chunk // 16, body, 0, unroll=8)

    pltpu.sync_copy(hcnt_v, cnt_hbm.at[wid])
    pltpu.sync_copy(hsum_v, sum_hbm.at[wid])


def _phase3_kernel(cnt0_ref, cnt1_ref, sum0_ref, sum1_ref, part_ref, np_ref, out_ref):
    total = 128 * _A
    npos = np_ref[0]
    k = jnp.maximum(jnp.minimum(npos * _SCALE_NEG, total - npos), 1)

    cnt = (jnp.sum(cnt0_ref[...], axis=0, keepdims=True)
           + jnp.sum(cnt1_ref[...], axis=0, keepdims=True))
    msum = (jnp.sum(sum0_ref[...], axis=0, keepdims=True)
            + jnp.sum(sum1_ref[...], axis=0, keepdims=True))
    liota = lax.broadcasted_iota(jnp.int32, (1, _NB), 1)

    def body(i, cur):
        t = cur + (jnp.int32(1) << (14 - i))
        s = jnp.sum(jnp.where(liota >= t, cnt, 0))
        return jnp.where(s >= k, t, cur)

    bstar = lax.fori_loop(0, 15, body, jnp.int32(0))

    cnt_above = jnp.sum(jnp.where(liota > bstar, cnt, 0))
    sum_above = jnp.sum(jnp.where(liota > bstar, msum, 0.0))
    edge = lax.bitcast_convert_type(bstar << 16, jnp.float32)
    neg_loss = sum_above + (k - cnt_above).astype(jnp.float32) * edge
    out_ref[0] = part_ref[0] + neg_loss


def _phase1_call(db, yp, tb, tbt, labf):
    Bh = yp.shape[0]
    A = yp.shape[2]
    return pl.pallas_call(
        _phase1_kernel,
        grid=(Bh // _BPP,),
        in_specs=[
            pl.BlockSpec((_DB_PACK.shape[0], A), lambda b: (0, 0)),
            pl.BlockSpec((_BPP, 25, A), lambda b: (b, 0, 0)),
            pl.BlockSpec((_BPP, 4, 32), lambda b: (b, 0, 0)),
            pl.BlockSpec((_BPP, 32, 4), lambda b: (b, 0, 0)),
            pl.BlockSpec((_BPP, 1, 32), lambda b: (b, 0, 0)),
        ],
        out_specs=[
            pl.BlockSpec((_BPP, 1, A), lambda b: (b, 0, 0)),
            pl.BlockSpec(memory_space=pltpu.SMEM),
            pl.BlockSpec(memory_space=pltpu.SMEM),
        ],
        out_shape=[
            jax.ShapeDtypeStruct((Bh, 1, A), jnp.int32),
            jax.ShapeDtypeStruct((1,), jnp.float32),
            jax.ShapeDtypeStruct((1,), jnp.int32),
        ],
    )(db, yp, tb, tbt, labf)


def _sc_hist_call(keys_flat):
    n = keys_flat.shape[0]
    sc_hist = pl.kernel(
        _sc_hist_kernel,
        mesh=plsc.VectorSubcoreMesh(core_axis_name="c", subcore_axis_name="s"),
        out_type=[
            jax.ShapeDtypeStruct((_SC_TILES, _NB), jnp.int32),
            jax.ShapeDtypeStruct((_SC_TILES, _NB), jnp.float32),
        ],
        scratch_types=[
            pltpu.VMEM((n // _SC_TILES,), jnp.int32),
            pltpu.VMEM((_NB,), jnp.int32),
            pltpu.VMEM((_NB,), jnp.float32),
            pltpu.SemaphoreType.DMA,
        ],
        compiler_params=pltpu.CompilerParams(needs_layout_passes=False),
    )
    return sc_hist(keys_flat)


def kernel(y_pred, true_boxes, true_labels):
    B, _, A = y_pred.shape
    db = jnp.asarray(_DB_PACK)
    tbt = jnp.transpose(true_boxes, (0, 2, 1))
    labf = true_labels.astype(jnp.float32).reshape(B, 1, 32)

    H = B // 2
    halves = []
    for lo in (0, H):
        keys, part, npos = _phase1_call(
            db,
            jax.lax.slice_in_dim(y_pred, lo, lo + H, axis=0),
            jax.lax.slice_in_dim(true_boxes, lo, lo + H, axis=0),
            jax.lax.slice_in_dim(tbt, lo, lo + H, axis=0),
            jax.lax.slice_in_dim(labf, lo, lo + H, axis=0),
        )
        n = H * A
        n_pad = ((n + 16 * _SC_TILES - 1) // (16 * _SC_TILES)) * (16 * _SC_TILES)
        keys_flat = jnp.pad(keys.reshape(n), (0, n_pad - n), constant_values=-1)
        halves.append((_sc_hist_call(keys_flat), part, npos))

    (cnt0, sum0), part0, np0 = halves[0]
    (cnt1, sum1), part1, np1 = halves[1]

    out = pl.pallas_call(
        _phase3_kernel,
        in_specs=[
            pl.BlockSpec((_SC_TILES, _NB), lambda: (0, 0)),
            pl.BlockSpec((_SC_TILES, _NB), lambda: (0, 0)),
            pl.BlockSpec((_SC_TILES, _NB), lambda: (0, 0)),
            pl.BlockSpec((_SC_TILES, _NB), lambda: (0, 0)),
            pl.BlockSpec(memory_space=pltpu.SMEM),
            pl.BlockSpec(memory_space=pltpu.SMEM),
        ],
        out_specs=pl.BlockSpec(memory_space=pltpu.SMEM),
        out_shape=jax.ShapeDtypeStruct((1,), jnp.float32),
    )(cnt0, cnt1, sum0, sum1, part0 + part1, np0 + np1)

    return out[0]


# speedup vs baseline: 1.3966x; 1.3966x over previous
from math import sqrt

import jax
import jax.numpy as jnp
import numpy as np
from jax import lax
from jax.experimental import pallas as pl
from jax.experimental.pallas import tpu as pltpu
from jax.experimental.pallas import tpu_sc as plsc

_MAPS_SIZE = [38, 19, 10, 5, 3, 1]
_NUM_ANCHORS = [4, 6, 6, 6, 4, 4]
_RATIOS = [1.0, 2.0, 0.5, 3.0, 1.0 / 3.0]
_THRESHOLD = 0.5
_SCALE_NEG = 3


def _dbox_scale(k, m=6, smin=0.2, smax=0.9):
    return smin + (smax - smin) * (k - 1) / (m - 1)


def _default_box_np():
    m = 6
    scales = [_dbox_scale(k) for k in range(1, m + 2)]
    scales_hat = [sqrt(scales[k] * scales[k + 1]) for k in range(m)]
    boxes = []
    for k in range(m):
        size = _MAPS_SIZE[k]
        coords = (np.arange(size, dtype=np.float32) + 0.5) / size
        cx = np.tile(coords[None, :], (size, 1)).reshape(-1)
        cy = np.tile(coords[:, None], (1, size)).reshape(-1)
        for idx in range(_NUM_ANCHORS[k] - 1):
            w = np.full_like(cx, scales[k] * sqrt(_RATIOS[idx]))
            h = np.full_like(cx, scales[k] / sqrt(_RATIOS[idx]))
            boxes.append(np.stack([cx, cy, w, h], axis=0))
        s = scales_hat[k]
        boxes.append(np.stack([cx, cy, np.full_like(cx, s), np.full_like(cx, s)], axis=0))
    return np.concatenate(boxes, axis=1).astype(np.float32)


_DB = _default_box_np()
_A = _DB.shape[1]

_Z = np.zeros_like(_DB[0])
_O = np.ones_like(_DB[0])
_DB_PACK = np.stack(
    [
        _DB[0] - _DB[2] / 2, _DB[1] - _DB[3] / 2,
        _DB[0] + _DB[2] / 2, _DB[1] + _DB[3] / 2,
        _DB[2] * _DB[3] + 1e-9,
        _DB[0], _DB[1], _Z, _Z,
        1.0 / _DB[2], 1.0 / _DB[3], _O, _O,
        _Z, _Z, -np.log(_DB[2]), -np.log(_DB[3]),
    ],
    axis=0,
).astype(np.float32)
_THIRD = np.float32(1.0 / 3.0)


_BPP = 2


def _phase1_kernel(db_ref, y_ref, tb_ref, tbt_ref, lab_ref, keys_ref, part_ref, np_ref):
    b = pl.program_id(0)

    def one_batch(i):
        return _one_batch(db_ref, y_ref, tb_ref, tbt_ref, lab_ref, keys_ref, i)

    part_b, num_pos_b = one_batch(0)
    for i in range(1, _BPP):
        p, n = one_batch(i)
        part_b = part_b + p
        num_pos_b = num_pos_b + n

    @pl.when(b == 0)
    def _():
        part_ref[0] = part_b
        np_ref[0] = num_pos_b

    @pl.when(b != 0)
    def _():
        part_ref[0] = part_ref[0] + part_b
        np_ref[0] = np_ref[0] + num_pos_b


def _one_batch(db_ref, y_ref, tb_ref, tbt_ref, lab_ref, keys_ref, i):
    db_l = db_ref[0:1, :]
    db_t = db_ref[1:2, :]
    db_r = db_ref[2:3, :]
    db_b = db_ref[3:4, :]
    db_area = db_ref[4:5, :]

    tbt = tbt_ref[i]
    gcx = tbt[:, 0:1]
    gcy = tbt[:, 1:2]
    gw = tbt[:, 2:3]
    gh = tbt[:, 3:4]
    gl = gcx - gw * 0.5
    gt = gcy - gh * 0.5
    gr = gcx + gw * 0.5
    gb = gcy + gh * 0.5
    g_area = gw * gh

    il = jnp.maximum(gl, db_l)
    it = jnp.maximum(gt, db_t)
    ir = jnp.minimum(gr, db_r)
    ib = jnp.minimum(gb, db_b)
    inter = jnp.maximum(ir - il, 0.0) * jnp.maximum(ib - it, 0.0)
    r = inter / (g_area + db_area)

    genc = 31 - jax.lax.broadcasted_iota(jnp.int32, (32, 1), 0)
    ki = (jax.lax.bitcast_convert_type(r, jnp.int32) & jnp.int32(~31)) | genc
    best = jnp.max(ki, axis=0, keepdims=True)

    pos = jax.lax.bitcast_convert_type(best & jnp.int32(~31), jnp.float32) > _THIRD
    posf = pos.astype(jnp.float32)
    num_pos_b = jnp.sum(pos.astype(jnp.int32))

    maskf = (ki == best).astype(jnp.float32)
    tb = tb_ref[i]
    logw = jnp.log(tb[2:3, :])
    logh = jnp.log(tb[3:4, :])
    labf = lab_ref[i]
    zeros3 = jnp.zeros((3, 32), jnp.float32)
    stacked = jnp.concatenate([tb[0:1], tb[1:2], logw, logh, labf, zeros3], axis=0)
    mm = jax.lax.dot_general(
        stacked, maskf, (((1,), (0,)), ((), ())),
        preferred_element_type=jnp.float32,
    )
    blab = mm[4:5]

    gh = (mm[0:4] - db_ref[5:9, :]) * db_ref[9:13, :] + db_ref[13:17, :]
    d = y_ref[i, 0:4, :] - gh
    ad = jnp.abs(d)
    sl1 = jnp.where(ad < 1.0, 0.5 * d * d, ad - 0.5)
    loc_loss = jnp.sum(sl1 * posf)

    cls = y_ref[i, 4:25, :]
    m = jnp.max(cls, axis=0, keepdims=True)
    s = jnp.sum(jnp.exp(cls - m), axis=0, keepdims=True)
    lse = m + jnp.log(s)

    ciota = jax.lax.broadcasted_iota(jnp.int32, (21, 1), 0).astype(jnp.float32)
    chmask = (blab == ciota).astype(jnp.float32) * posf
    sel_sum = jnp.sum(chmask * cls)
    pos_loss = jnp.sum(lse * posf) - sel_sum

    neg = lse - y_ref[i, 4:5, :]
    keys = jnp.where(pos, jnp.int32(-1), jax.lax.bitcast_convert_type(neg, jnp.int32))
    keys_ref[i] = keys

    return loc_loss + pos_loss, num_pos_b


_NB = 32768
_SC_TILES = 32


def _sc_hist_kernel(keys_hbm, cnt_hbm, sum_hbm, chunk_v, hcnt_v, hsum_v, sem):
    wid = lax.axis_index("s") * 2 + lax.axis_index("c")
    chunk = keys_hbm.shape[0] // _SC_TILES
    cp = pltpu.async_copy(keys_hbm.at[pl.ds(wid * chunk, chunk)], chunk_v, sem)

    zi = jnp.zeros((16,), jnp.int32)
    zf = jnp.zeros((16,), jnp.float32)

    def zbody(j, c):
        hcnt_v[pl.ds(j * 16, 16)] = zi
        hsum_v[pl.ds(j * 16, 16)] = zf
        return c

    lax.fori_loop(0, _NB // 16, zbody, 0, unroll=8)
    cp.wait()

    ones = jnp.ones((16,), jnp.int32)

    def body(j, c):
        k16 = chunk_v[pl.ds(j * 16, 16)]
        msk = k16 >= 0
        bkt = lax.shift_right_logical(jnp.maximum(k16, 0), 16)
        plsc.addupdate_scatter(hcnt_v, [bkt], ones, mask=msk)
        plsc.addupdate_scatter(hsum_v, [bkt], plsc.bitcast(k16, jnp.float32), mask=msk)
        return c

    lax.fori_loop(0, chunk // 16, body, 0, unroll=8)

    pltpu.sync_copy(hcnt_v, cnt_hbm.at[wid])
    pltpu.sync_copy(hsum_v, sum_hbm.at[wid])


def _phase3_kernel(cnt_ref, sum_ref, part_ref, np_ref, out_ref):
    total = 128 * _A
    npos = np_ref[0]
    k = jnp.maximum(jnp.minimum(npos * _SCALE_NEG, total - npos), 1)

    cnt = jnp.sum(cnt_ref[...], axis=0, keepdims=True)
    msum = jnp.sum(sum_ref[...], axis=0, keepdims=True)
    liota = lax.broadcasted_iota(jnp.int32, (1, _NB), 1)

    def body(i, cur):
        t = cur + (jnp.int32(1) << (14 - i))
        s = jnp.sum(jnp.where(liota >= t, cnt, 0))
        return jnp.where(s >= k, t, cur)

    bstar = lax.fori_loop(0, 15, body, jnp.int32(0))

    cnt_above = jnp.sum(jnp.where(liota > bstar, cnt, 0))
    sum_above = jnp.sum(jnp.where(liota > bstar, msum, 0.0))
    edge = lax.bitcast_convert_type(bstar << 16, jnp.float32)
    neg_loss = sum_above + (k - cnt_above).astype(jnp.float32) * edge
    out_ref[0] = part_ref[0] + neg_loss


def _phase1_call(db, yp, tb, tbt, labf):
    Bh = yp.shape[0]
    A = yp.shape[2]
    return pl.pallas_call(
        _phase1_kernel,
        grid=(Bh // _BPP,),
        in_specs=[
            pl.BlockSpec((_DB_PACK.shape[0], A), lambda b: (0, 0)),
            pl.BlockSpec((_BPP, 25, A), lambda b: (b, 0, 0)),
            pl.BlockSpec((_BPP, 4, 32), lambda b: (b, 0, 0)),
            pl.BlockSpec((_BPP, 32, 4), lambda b: (b, 0, 0)),
            pl.BlockSpec((_BPP, 1, 32), lambda b: (b, 0, 0)),
        ],
        out_specs=[
            pl.BlockSpec((_BPP, 1, A), lambda b: (b, 0, 0)),
            pl.BlockSpec(memory_space=pltpu.SMEM),
            pl.BlockSpec(memory_space=pltpu.SMEM),
        ],
        out_shape=[
            jax.ShapeDtypeStruct((Bh, 1, A), jnp.int32),
            jax.ShapeDtypeStruct((1,), jnp.float32),
            jax.ShapeDtypeStruct((1,), jnp.int32),
        ],
    )(db, yp, tb, tbt, labf)


def _sc_hist_call(keys_flat):
    n = keys_flat.shape[0]
    sc_hist = pl.kernel(
        _sc_hist_kernel,
        mesh=plsc.VectorSubcoreMesh(core_axis_name="c", subcore_axis_name="s"),
        out_type=[
            jax.ShapeDtypeStruct((_SC_TILES, _NB), jnp.int32),
            jax.ShapeDtypeStruct((_SC_TILES, _NB), jnp.float32),
        ],
        scratch_types=[
            pltpu.VMEM((n // _SC_TILES,), jnp.int32),
            pltpu.VMEM((_NB,), jnp.int32),
            pltpu.VMEM((_NB,), jnp.float32),
            pltpu.SemaphoreType.DMA,
        ],
        compiler_params=pltpu.CompilerParams(needs_layout_passes=False),
    )
    return sc_hist(keys_flat)


def kernel(y_pred, true_boxes, true_labels):
    B, _, A = y_pred.shape
    db = jnp.asarray(_DB_PACK)
    tbt = jnp.transpose(true_boxes, (0, 2, 1))
    labf = true_labels.astype(jnp.float32).reshape(B, 1, 32)

    keys, part, npos = _phase1_call(db, y_pred, true_boxes, tbt, labf)
    cnt, vsum = _sc_hist_call(keys.reshape(B * A))

    out = pl.pallas_call(
        _phase3_kernel,
        in_specs=[
            pl.BlockSpec((_SC_TILES, _NB), lambda: (0, 0)),
            pl.BlockSpec((_SC_TILES, _NB), lambda: (0, 0)),
            pl.BlockSpec(memory_space=pltpu.SMEM),
            pl.BlockSpec(memory_space=pltpu.SMEM),
        ],
        out_specs=pl.BlockSpec(memory_space=pltpu.SMEM),
        out_shape=jax.ShapeDtypeStruct((1,), jnp.float32),
    )(cnt, vsum, part, npos)

    return out[0]


# speedup vs baseline: 1.4254x; 1.0206x over previous
from math import sqrt

import jax
import jax.numpy as jnp
import numpy as np
from jax import lax
from jax.experimental import pallas as pl
from jax.experimental.pallas import tpu as pltpu
from jax.experimental.pallas import tpu_sc as plsc

_MAPS_SIZE = [38, 19, 10, 5, 3, 1]
_NUM_ANCHORS = [4, 6, 6, 6, 4, 4]
_RATIOS = [1.0, 2.0, 0.5, 3.0, 1.0 / 3.0]
_THRESHOLD = 0.5
_SCALE_NEG = 3


def _dbox_scale(k, m=6, smin=0.2, smax=0.9):
    return smin + (smax - smin) * (k - 1) / (m - 1)


def _default_box_np():
    m = 6
    scales = [_dbox_scale(k) for k in range(1, m + 2)]
    scales_hat = [sqrt(scales[k] * scales[k + 1]) for k in range(m)]
    boxes = []
    for k in range(m):
        size = _MAPS_SIZE[k]
        coords = (np.arange(size, dtype=np.float32) + 0.5) / size
        cx = np.tile(coords[None, :], (size, 1)).reshape(-1)
        cy = np.tile(coords[:, None], (1, size)).reshape(-1)
        for idx in range(_NUM_ANCHORS[k] - 1):
            w = np.full_like(cx, scales[k] * sqrt(_RATIOS[idx]))
            h = np.full_like(cx, scales[k] / sqrt(_RATIOS[idx]))
            boxes.append(np.stack([cx, cy, w, h], axis=0))
        s = scales_hat[k]
        boxes.append(np.stack([cx, cy, np.full_like(cx, s), np.full_like(cx, s)], axis=0))
    return np.concatenate(boxes, axis=1).astype(np.float32)


_DB = _default_box_np()
_A = _DB.shape[1]

_Z = np.zeros_like(_DB[0])
_O = np.ones_like(_DB[0])
_DB_PACK = np.stack(
    [
        _DB[0] - _DB[2] / 2, _DB[1] - _DB[3] / 2,
        _DB[0] + _DB[2] / 2, _DB[1] + _DB[3] / 2,
        _DB[2] * _DB[3] + 1e-9,
        _DB[0], _DB[1], _Z, _Z,
        1.0 / _DB[2], 1.0 / _DB[3], _O, _O,
        _Z, _Z, -np.log(_DB[2]), -np.log(_DB[3]),
    ],
    axis=0,
).astype(np.float32)
_THIRD = np.float32(1.0 / 3.0)


_BPP = 4


def _phase1_kernel(db_ref, y_ref, tb_ref, tbt_ref, lab_ref, keys_ref, part_ref, np_ref):
    b = pl.program_id(0)

    def one_batch(i):
        return _one_batch(db_ref, y_ref, tb_ref, tbt_ref, lab_ref, keys_ref, i)

    part_b, num_pos_b = one_batch(0)
    for i in range(1, _BPP):
        p, n = one_batch(i)
        part_b = part_b + p
        num_pos_b = num_pos_b + n

    @pl.when(b == 0)
    def _():
        part_ref[0] = part_b
        np_ref[0] = num_pos_b

    @pl.when(b != 0)
    def _():
        part_ref[0] = part_ref[0] + part_b
        np_ref[0] = np_ref[0] + num_pos_b


def _one_batch(db_ref, y_ref, tb_ref, tbt_ref, lab_ref, keys_ref, i):
    db_l = db_ref[0:1, :]
    db_t = db_ref[1:2, :]
    db_r = db_ref[2:3, :]
    db_b = db_ref[3:4, :]
    db_area = db_ref[4:5, :]

    tbt = tbt_ref[i]
    gcx = tbt[:, 0:1]
    gcy = tbt[:, 1:2]
    gw = tbt[:, 2:3]
    gh = tbt[:, 3:4]
    gl = gcx - gw * 0.5
    gt = gcy - gh * 0.5
    gr = gcx + gw * 0.5
    gb = gcy + gh * 0.5
    g_area = gw * gh

    il = jnp.maximum(gl, db_l)
    it = jnp.maximum(gt, db_t)
    ir = jnp.minimum(gr, db_r)
    ib = jnp.minimum(gb, db_b)
    inter = jnp.maximum(ir - il, 0.0) * jnp.maximum(ib - it, 0.0)
    r = inter / (g_area + db_area)

    genc = 31 - jax.lax.broadcasted_iota(jnp.int32, (32, 1), 0)
    ki = (jax.lax.bitcast_convert_type(r, jnp.int32) & jnp.int32(~31)) | genc
    best = jnp.max(ki, axis=0, keepdims=True)

    pos = jax.lax.bitcast_convert_type(best & jnp.int32(~31), jnp.float32) > _THIRD
    posf = pos.astype(jnp.float32)
    num_pos_b = jnp.sum(pos.astype(jnp.int32))

    maskf = (ki == best).astype(jnp.float32)
    tb = tb_ref[i]
    logw = jnp.log(tb[2:3, :])
    logh = jnp.log(tb[3:4, :])
    labf = lab_ref[i]
    zeros3 = jnp.zeros((3, 32), jnp.float32)
    stacked = jnp.concatenate([tb[0:1], tb[1:2], logw, logh, labf, zeros3], axis=0)
    mm = jax.lax.dot_general(
        stacked, maskf, (((1,), (0,)), ((), ())),
        preferred_element_type=jnp.float32,
    )
    blab = mm[4:5]

    gh = (mm[0:4] - db_ref[5:9, :]) * db_ref[9:13, :] + db_ref[13:17, :]
    d = y_ref[i, 0:4, :] - gh
    ad = jnp.abs(d)
    sl1 = jnp.where(ad < 1.0, 0.5 * d * d, ad - 0.5)
    loc_loss = jnp.sum(sl1 * posf)

    cls = y_ref[i, 4:25, :]
    m = jnp.max(cls, axis=0, keepdims=True)
    s = jnp.sum(jnp.exp(cls - m), axis=0, keepdims=True)
    lse = m + jnp.log(s)

    ciota = jax.lax.broadcasted_iota(jnp.int32, (21, 1), 0).astype(jnp.float32)
    chmask = (blab == ciota).astype(jnp.float32) * posf
    sel_sum = jnp.sum(chmask * cls)
    pos_loss = jnp.sum(lse * posf) - sel_sum

    neg = lse - y_ref[i, 4:5, :]
    keys = jnp.where(pos, jnp.int32(-1), jax.lax.bitcast_convert_type(neg, jnp.int32))
    keys_ref[i] = keys

    return loc_loss + pos_loss, num_pos_b


_NB = 32768
_SC_TILES = 32


def _sc_hist_kernel(keys_hbm, cnt_hbm, sum_hbm, chunk_v, hcnt_v, hsum_v, sem):
    wid = lax.axis_index("s") * 2 + lax.axis_index("c")
    chunk = keys_hbm.shape[0] // _SC_TILES
    cp = pltpu.async_copy(keys_hbm.at[pl.ds(wid * chunk, chunk)], chunk_v, sem)

    zi = jnp.zeros((16,), jnp.int32)
    zf = jnp.zeros((16,), jnp.float32)

    def zbody(j, c):
        hcnt_v[pl.ds(j * 16, 16)] = zi
        hsum_v[pl.ds(j * 16, 16)] = zf
        return c

    lax.fori_loop(0, _NB // 16, zbody, 0, unroll=8)
    cp.wait()

    ones = jnp.ones((16,), jnp.int32)

    def body(j, c):
        k16 = chunk_v[pl.ds(j * 16, 16)]
        msk = k16 >= 0
        bkt = lax.shift_right_logical(jnp.maximum(k16, 0), 16)
        plsc.addupdate_scatter(hcnt_v, [bkt], ones, mask=msk)
        plsc.addupdate_scatter(hsum_v, [bkt], plsc.bitcast(k16, jnp.float32), mask=msk)
        return c

    lax.fori_loop(0, chunk // 16, body, 0, unroll=8)

    pltpu.sync_copy(hcnt_v, cnt_hbm.at[wid])
    pltpu.sync_copy(hsum_v, sum_hbm.at[wid])


def _phase3_kernel(cnt_ref, sum_ref, part_ref, np_ref, out_ref):
    total = 128 * _A
    npos = np_ref[0]
    k = jnp.maximum(jnp.minimum(npos * _SCALE_NEG, total - npos), 1)

    cnt = jnp.sum(cnt_ref[...], axis=0, keepdims=True)
    msum = jnp.sum(sum_ref[...], axis=0, keepdims=True)
    liota = lax.broadcasted_iota(jnp.int32, (1, _NB), 1)

    def body(i, cur):
        t = cur + (jnp.int32(1) << (14 - i))
        s = jnp.sum(jnp.where(liota >= t, cnt, 0))
        return jnp.where(s >= k, t, cur)

    bstar = lax.fori_loop(0, 15, body, jnp.int32(0))

    cnt_above = jnp.sum(jnp.where(liota > bstar, cnt, 0))
    sum_above = jnp.sum(jnp.where(liota > bstar, msum, 0.0))
    edge = lax.bitcast_convert_type(bstar << 16, jnp.float32)
    neg_loss = sum_above + (k - cnt_above).astype(jnp.float32) * edge
    out_ref[0] = part_ref[0] + neg_loss


def _phase1_call(db, yp, tb, tbt, labf):
    Bh = yp.shape[0]
    A = yp.shape[2]
    return pl.pallas_call(
        _phase1_kernel,
        grid=(Bh // _BPP,),
        in_specs=[
            pl.BlockSpec((_DB_PACK.shape[0], A), lambda b: (0, 0)),
            pl.BlockSpec((_BPP, 25, A), lambda b: (b, 0, 0)),
            pl.BlockSpec((_BPP, 4, 32), lambda b: (b, 0, 0)),
            pl.BlockSpec((_BPP, 32, 4), lambda b: (b, 0, 0)),
            pl.BlockSpec((_BPP, 1, 32), lambda b: (b, 0, 0)),
        ],
        out_specs=[
            pl.BlockSpec((_BPP, 1, A), lambda b: (b, 0, 0)),
            pl.BlockSpec(memory_space=pltpu.SMEM),
            pl.BlockSpec(memory_space=pltpu.SMEM),
        ],
        out_shape=[
            jax.ShapeDtypeStruct((Bh, 1, A), jnp.int32),
            jax.ShapeDtypeStruct((1,), jnp.float32),
            jax.ShapeDtypeStruct((1,), jnp.int32),
        ],
    )(db, yp, tb, tbt, labf)


def _sc_hist_call(keys_flat):
    n = keys_flat.shape[0]
    sc_hist = pl.kernel(
        _sc_hist_kernel,
        mesh=plsc.VectorSubcoreMesh(core_axis_name="c", subcore_axis_name="s"),
        out_type=[
            jax.ShapeDtypeStruct((_SC_TILES, _NB), jnp.int32),
            jax.ShapeDtypeStruct((_SC_TILES, _NB), jnp.float32),
        ],
        scratch_types=[
            pltpu.VMEM((n // _SC_TILES,), jnp.int32),
            pltpu.VMEM((_NB,), jnp.int32),
            pltpu.VMEM((_NB,), jnp.float32),
            pltpu.SemaphoreType.DMA,
        ],
        compiler_params=pltpu.CompilerParams(needs_layout_passes=False),
    )
    return sc_hist(keys_flat)


def kernel(y_pred, true_boxes, true_labels):
    B, _, A = y_pred.shape
    db = jnp.asarray(_DB_PACK)
    tbt = jnp.transpose(true_boxes, (0, 2, 1))
    labf = true_labels.astype(jnp.float32).reshape(B, 1, 32)

    keys, part, npos = _phase1_call(db, y_pred, true_boxes, tbt, labf)
    cnt, vsum = _sc_hist_call(keys.reshape(B * A))

    out = pl.pallas_call(
        _phase3_kernel,
        in_specs=[
            pl.BlockSpec((_SC_TILES, _NB), lambda: (0, 0)),
            pl.BlockSpec((_SC_TILES, _NB), lambda: (0, 0)),
            pl.BlockSpec(memory_space=pltpu.SMEM),
            pl.BlockSpec(memory_space=pltpu.SMEM),
        ],
        out_specs=pl.BlockSpec(memory_space=pltpu.SMEM),
        out_shape=jax.ShapeDtypeStruct((1,), jnp.float32),
    )(cnt, vsum, part, npos)

    return out[0]


# speedup vs baseline: 1.4255x; 1.0000x over previous
from math import sqrt

import jax
import jax.numpy as jnp
import numpy as np
from jax import lax
from jax.experimental import pallas as pl
from jax.experimental.pallas import tpu as pltpu
from jax.experimental.pallas import tpu_sc as plsc

_MAPS_SIZE = [38, 19, 10, 5, 3, 1]
_NUM_ANCHORS = [4, 6, 6, 6, 4, 4]
_RATIOS = [1.0, 2.0, 0.5, 3.0, 1.0 / 3.0]
_SCALE_NEG = 3


def _dbox_scale(k, m=6, smin=0.2, smax=0.9):
    return smin + (smax - smin) * (k - 1) / (m - 1)


def _default_box_np():
    m = 6
    scales = [_dbox_scale(k) for k in range(1, m + 2)]
    scales_hat = [sqrt(scales[k] * scales[k + 1]) for k in range(m)]
    boxes = []
    for k in range(m):
        size = _MAPS_SIZE[k]
        coords = (np.arange(size, dtype=np.float32) + 0.5) / size
        cx = np.tile(coords[None, :], (size, 1)).reshape(-1)
        cy = np.tile(coords[:, None], (1, size)).reshape(-1)
        for idx in range(_NUM_ANCHORS[k] - 1):
            w = np.full_like(cx, scales[k] * sqrt(_RATIOS[idx]))
            h = np.full_like(cx, scales[k] / sqrt(_RATIOS[idx]))
            boxes.append(np.stack([cx, cy, w, h], axis=0))
        s = scales_hat[k]
        boxes.append(np.stack([cx, cy, np.full_like(cx, s), np.full_like(cx, s)], axis=0))
    return np.concatenate(boxes, axis=1).astype(np.float32)


_DB = _default_box_np()
_A = _DB.shape[1]

_Z = np.zeros_like(_DB[0])
_O = np.ones_like(_DB[0])
_DB_PACK = np.stack(
    [
        _DB[0] - _DB[2] / 2, _DB[1] - _DB[3] / 2,
        _DB[0] + _DB[2] / 2, _DB[1] + _DB[3] / 2,
        _DB[2] * _DB[3] + 1e-9,
        _DB[0], _DB[1], _Z, _Z,
        1.0 / _DB[2], 1.0 / _DB[3], _O, _O,
        _Z, _Z, -np.log(_DB[2]), -np.log(_DB[3]),
    ],
    axis=0,
).astype(np.float32)
_THIRD = np.float32(1.0 / 3.0)


_BPP = 4


def _phase1_kernel(db_ref, y_ref, tb_ref, tbt_ref, lab_ref, keys_ref, part_ref, np_ref):
    b = pl.program_id(0)

    def one_batch(i):
        return _one_batch(db_ref, y_ref, tb_ref, tbt_ref, lab_ref, keys_ref, i)

    part_b, num_pos_b = one_batch(0)
    for i in range(1, _BPP):
        p, n = one_batch(i)
        part_b = part_b + p
        num_pos_b = num_pos_b + n

    @pl.when(b == 0)
    def _():
        part_ref[0] = part_b
        np_ref[0] = num_pos_b

    @pl.when(b != 0)
    def _():
        part_ref[0] = part_ref[0] + part_b
        np_ref[0] = np_ref[0] + num_pos_b


def _one_batch(db_ref, y_ref, tb_ref, tbt_ref, lab_ref, keys_ref, i):
    db_l = db_ref[0:1, :]
    db_t = db_ref[1:2, :]
    db_r = db_ref[2:3, :]
    db_b = db_ref[3:4, :]
    db_area = db_ref[4:5, :]

    tbt = tbt_ref[i]
    gcx = tbt[:, 0:1]
    gcy = tbt[:, 1:2]
    gw = tbt[:, 2:3]
    gh = tbt[:, 3:4]
    gl = gcx - gw * 0.5
    gt = gcy - gh * 0.5
    gr = gcx + gw * 0.5
    gb = gcy + gh * 0.5
    g_area = gw * gh

    il = jnp.maximum(gl, db_l)
    it = jnp.maximum(gt, db_t)
    ir = jnp.minimum(gr, db_r)
    ib = jnp.minimum(gb, db_b)
    inter = jnp.maximum(ir - il, 0.0) * jnp.maximum(ib - it, 0.0)
    r = inter / (g_area + db_area)

    genc = 31 - jax.lax.broadcasted_iota(jnp.int32, (32, 1), 0)
    ki = (jax.lax.bitcast_convert_type(r, jnp.int32) & jnp.int32(~31)) | genc
    best = jnp.max(ki, axis=0, keepdims=True)

    pos = jax.lax.bitcast_convert_type(best & jnp.int32(~31), jnp.float32) > _THIRD
    posf = pos.astype(jnp.float32)
    num_pos_b = jnp.sum(pos.astype(jnp.int32))

    maskf = (ki == best).astype(jnp.float32)
    tb = tb_ref[i]
    logw = jnp.log(tb[2:3, :])
    logh = jnp.log(tb[3:4, :])
    labf = lab_ref[i]
    zeros3 = jnp.zeros((3, 32), jnp.float32)
    stacked = jnp.concatenate([tb[0:1], tb[1:2], logw, logh, labf, zeros3], axis=0)
    mm = jax.lax.dot_general(
        stacked, maskf, (((1,), (0,)), ((), ())),
        preferred_element_type=jnp.float32,
    )
    blab = mm[4:5]

    gh = (mm[0:4] - db_ref[5:9, :]) * db_ref[9:13, :] + db_ref[13:17, :]
    d = y_ref[i, 0:4, :] - gh
    ad = jnp.abs(d)
    sl1 = jnp.where(ad < 1.0, 0.5 * d * d, ad - 0.5)
    loc_loss = jnp.sum(sl1 * posf)

    cls = y_ref[i, 4:25, :]
    m = jnp.max(cls, axis=0, keepdims=True)
    s = jnp.sum(jnp.exp(cls - m), axis=0, keepdims=True)
    lse = m + jnp.log(s)

    ciota = jax.lax.broadcasted_iota(jnp.int32, (21, 1), 0).astype(jnp.float32)
    chmask = (blab == ciota).astype(jnp.float32) * posf
    sel_sum = jnp.sum(chmask * cls)
    pos_loss = jnp.sum(lse * posf) - sel_sum

    neg = lse - y_ref[i, 4:5, :]
    keys = jnp.where(pos, jnp.int32(-1), jax.lax.bitcast_convert_type(neg, jnp.int32))
    keys_ref[i] = keys

    return loc_loss + pos_loss, num_pos_b


_NB = 32768
_SC_TILES = 32


def _sc_hist_kernel(keys_hbm, cnt_hbm, sum_hbm, chunk_v, hcnt_v, hsum_v, sem):
    wid = lax.axis_index("s") * 2 + lax.axis_index("c")
    chunk = keys_hbm.shape[0] // _SC_TILES
    cp = pltpu.async_copy(keys_hbm.at[pl.ds(wid * chunk, chunk)], chunk_v, sem)

    zi = jnp.zeros((16,), jnp.int32)
    zf = jnp.zeros((16,), jnp.float32)

    def zbody(j, c):
        hcnt_v[pl.ds(j * 16, 16)] = zi
        hsum_v[pl.ds(j * 16, 16)] = zf
        return c

    lax.fori_loop(0, _NB // 16, zbody, 0, unroll=8)
    cp.wait()

    ones = jnp.ones((16,), jnp.int32)

    def body(j, c):
        k16 = chunk_v[pl.ds(j * 16, 16)]
        msk = k16 >= 0
        bkt = lax.shift_right_logical(jnp.maximum(k16, 0), 16)
        plsc.addupdate_scatter(hcnt_v, [bkt], ones, mask=msk)
        plsc.addupdate_scatter(hsum_v, [bkt], plsc.bitcast(k16, jnp.float32), mask=msk)
        return c

    lax.fori_loop(0, chunk // 16, body, 0, unroll=16)

    pltpu.sync_copy(hcnt_v, cnt_hbm.at[wid])
    pltpu.sync_copy(hsum_v, sum_hbm.at[wid])


def _phase3_kernel(cnt_ref, sum_ref, part_ref, np_ref, out_ref):
    total = 128 * _A
    npos = np_ref[0]
    k = jnp.maximum(jnp.minimum(npos * _SCALE_NEG, total - npos), 1)

    cnt = jnp.sum(cnt_ref[...], axis=0, keepdims=True)
    msum = jnp.sum(sum_ref[...], axis=0, keepdims=True)
    liota = lax.broadcasted_iota(jnp.int32, (1, _NB), 1)

    def body(i, cur):
        t = cur + (jnp.int32(1) << (14 - i))
        s = jnp.sum(jnp.where(liota >= t, cnt, 0))
        return jnp.where(s >= k, t, cur)

    bstar = lax.fori_loop(0, 15, body, jnp.int32(0))

    cnt_above = jnp.sum(jnp.where(liota > bstar, cnt, 0))
    sum_above = jnp.sum(jnp.where(liota > bstar, msum, 0.0))
    edge = lax.bitcast_convert_type(bstar << 16, jnp.float32)
    neg_loss = sum_above + (k - cnt_above).astype(jnp.float32) * edge
    out_ref[0] = part_ref[0] + neg_loss


def _phase1_call(db, yp, tb, tbt, labf):
    Bh = yp.shape[0]
    A = yp.shape[2]
    return pl.pallas_call(
        _phase1_kernel,
        grid=(Bh // _BPP,),
        in_specs=[
            pl.BlockSpec((_DB_PACK.shape[0], A), lambda b: (0, 0)),
            pl.BlockSpec((_BPP, 25, A), lambda b: (b, 0, 0)),
            pl.BlockSpec((_BPP, 4, 32), lambda b: (b, 0, 0)),
            pl.BlockSpec((_BPP, 32, 4), lambda b: (b, 0, 0)),
            pl.BlockSpec((_BPP, 1, 32), lambda b: (b, 0, 0)),
        ],
        out_specs=[
            pl.BlockSpec((_BPP, 1, A), lambda b: (b, 0, 0)),
            pl.BlockSpec(memory_space=pltpu.SMEM),
            pl.BlockSpec(memory_space=pltpu.SMEM),
        ],
        out_shape=[
            jax.ShapeDtypeStruct((Bh, 1, A), jnp.int32),
            jax.ShapeDtypeStruct((1,), jnp.float32),
            jax.ShapeDtypeStruct((1,), jnp.int32),
        ],
    )(db, yp, tb, tbt, labf)


def _sc_hist_call(keys_flat):
    n = keys_flat.shape[0]
    sc_hist = pl.kernel(
        _sc_hist_kernel,
        mesh=plsc.VectorSubcoreMesh(core_axis_name="c", subcore_axis_name="s"),
        out_type=[
            jax.ShapeDtypeStruct((_SC_TILES, _NB), jnp.int32),
            jax.ShapeDtypeStruct((_SC_TILES, _NB), jnp.float32),
        ],
        scratch_types=[
            pltpu.VMEM((n // _SC_TILES,), jnp.int32),
            pltpu.VMEM((_NB,), jnp.int32),
            pltpu.VMEM((_NB,), jnp.float32),
            pltpu.SemaphoreType.DMA,
        ],
        compiler_params=pltpu.CompilerParams(needs_layout_passes=False),
    )
    return sc_hist(keys_flat)


def kernel(y_pred, true_boxes, true_labels):
    B, _, A = y_pred.shape
    db = jnp.asarray(_DB_PACK)
    tbt = jnp.transpose(true_boxes, (0, 2, 1))
    labf = true_labels.astype(jnp.float32).reshape(B, 1, 32)

    keys, part, npos = _phase1_call(db, y_pred, true_boxes, tbt, labf)
    cnt, vsum = _sc_hist_call(keys.reshape(B * A))

    out = pl.pallas_call(
        _phase3_kernel,
        in_specs=[
            pl.BlockSpec((_SC_TILES, _NB), lambda: (0, 0)),
            pl.BlockSpec((_SC_TILES, _NB), lambda: (0, 0)),
            pl.BlockSpec(memory_space=pltpu.SMEM),
            pl.BlockSpec(memory_space=pltpu.SMEM),
        ],
        out_specs=pl.BlockSpec(memory_space=pltpu.SMEM),
        out_shape=jax.ShapeDtypeStruct((1,), jnp.float32),
    )(cnt, vsum, part, npos)

    return out[0]
